# Initial kernel scaffold; baseline (speedup 1.0000x reference)
import jax
import jax.numpy as jnp
from jax.experimental import pallas as pl


def kernel(z, embedding):
    raise NotImplementedError("write your pallas kernel here")


# speedup vs baseline: 3.2707x; 3.2707x over previous
import jax
import jax.numpy as jnp
from jax.experimental import pallas as pl

_ROWS = 1024


def _vq_block(z_ref, emb_ref, out_ref):
    z = z_ref[...]
    emb = emb_ref[...]
    ncodes = emb.shape[0]
    dot = jax.lax.dot_general(
        z.astype(jnp.bfloat16), emb.astype(jnp.bfloat16),
        (((1,), (1,)), ((), ())),
        preferred_element_type=jnp.float32,
    )
    z2 = jnp.sum(z * z, axis=1, keepdims=True)
    e2 = jnp.sum(emb * emb, axis=1)
    dist = z2 + e2 - 2.0 * dot
    m = jnp.min(dist, axis=1, keepdims=True)
    iota = jax.lax.broadcasted_iota(jnp.int32, dist.shape, 1)
    idx = jnp.min(jnp.where(dist == m, iota, ncodes), axis=1, keepdims=True)
    one_hot = (iota == idx).astype(jnp.bfloat16)
    out_ref[...] = jax.lax.dot_general(
        one_hot, emb.astype(jnp.bfloat16), (((1,), (0,)), ((), ())),
        preferred_element_type=jnp.float32,
    )


def kernel(z, embedding):
    n, d = z.shape
    return pl.pallas_call(
        _vq_block,
        grid=(n // _ROWS,),
        in_specs=[
            pl.BlockSpec((_ROWS, d), lambda i: (i, 0)),
            pl.BlockSpec(embedding.shape, lambda i: (0, 0)),
        ],
        out_specs=pl.BlockSpec((_ROWS, d), lambda i: (i, 0)),
        out_shape=jax.ShapeDtypeStruct((n, d), z.dtype),
    )(z, embedding)


# speedup vs baseline: 3.8869x; 1.1884x over previous
import functools

import jax
import jax.numpy as jnp
from jax import lax
from jax.experimental import pallas as pl
from jax.experimental.pallas import tpu as pltpu
from jax.experimental.pallas import tpu_sc as plsc

_COLS = 1024


def _argmin_block(e2_ref, z2_ref, emb_ref, z_ref, idx_ref):
    emb = emb_ref[...]
    z = z_ref[...]
    dot_t = lax.dot_general(
        emb.astype(jnp.bfloat16), z.astype(jnp.bfloat16),
        (((1,), (1,)), ((), ())), preferred_element_type=jnp.float32,
    )
    dist = (z2_ref[...] + e2_ref[...]) - 2.0 * dot_t
    m = jnp.min(dist, axis=0, keepdims=True)
    iota = lax.broadcasted_iota(jnp.int32, dist.shape, 0)
    idx = jnp.min(jnp.where(dist == m, iota, dist.shape[0]),
                  axis=0, keepdims=True)
    idx_ref[...] = idx.reshape(1, 1, idx.shape[1])


def _compute_indices(z, embedding):
    n, d = z.shape
    ncodes = embedding.shape[0]
    z2 = jnp.sum(z ** 2, axis=1)[None, :]
    e2 = jnp.sum(embedding ** 2, axis=1)[:, None]
    idx3 = pl.pallas_call(
        _argmin_block,
        grid=(n // _COLS,),
        in_specs=[
            pl.BlockSpec((ncodes, 1), lambda i: (0, 0)),
            pl.BlockSpec((1, _COLS), lambda i: (0, i)),
            pl.BlockSpec((ncodes, d), lambda i: (0, 0)),
            pl.BlockSpec((_COLS, d), lambda i: (i, 0)),
        ],
        out_specs=pl.BlockSpec((1, 1, _COLS), lambda i: (i, 0, 0)),
        out_shape=jax.ShapeDtypeStruct((n // _COLS, 1, _COLS), jnp.int32),
    )(e2, z2, embedding, z)
    return idx3.reshape(n)


@functools.lru_cache(maxsize=None)
def _make_gather(n, d, b_per_w):
    mesh = plsc.VectorSubcoreMesh(core_axis_name="c", subcore_axis_name="s")

    @functools.partial(
        pl.kernel, mesh=mesh,
        out_type=jax.ShapeDtypeStruct((n, d), jnp.float32),
        compiler_params=pltpu.CompilerParams(use_tc_tiling_on_sc=False),
        scratch_types=[
            pltpu.VMEM((b_per_w,), jnp.int32),
            pltpu.VMEM((b_per_w, d), jnp.float32),
            pltpu.SemaphoreType.DMA,
        ],
    )
    def gather(table_hbm, idx_hbm, out_hbm, idx_v, rows_v, sem):
        wid = lax.axis_index("s") * 2 + lax.axis_index("c")
        base = wid * b_per_w
        pltpu.sync_copy(idx_hbm.at[pl.ds(base, b_per_w)], idx_v)
        pltpu.async_copy(table_hbm.at[idx_v], rows_v, sem).wait()
        pltpu.sync_copy(rows_v, out_hbm.at[pl.ds(base, b_per_w)])

    return gather


def kernel(z, embedding):
    n, d = z.shape
    idx = _compute_indices(z, embedding)
    table = embedding.astype(jnp.bfloat16).astype(jnp.float32)
    return _make_gather(n, d, n // 32)(table, idx)


# speedup vs baseline: 4.1239x; 1.0610x over previous
import functools

import jax
import jax.numpy as jnp
from jax import lax
from jax.experimental import pallas as pl
from jax.experimental.pallas import tpu as pltpu
from jax.experimental.pallas import tpu_sc as plsc

_COLS = 1024


def _argmin_block(e2_ref, z2_ref, iota_ref, emb_ref, z_ref, idx_ref):
    emb = emb_ref[...]
    z = z_ref[...]
    zb = z.astype(jnp.bfloat16) * jnp.bfloat16(-2.0)
    dot_m2 = lax.dot_general(
        emb.astype(jnp.bfloat16), zb,
        (((1,), (1,)), ((), ())), preferred_element_type=jnp.float32,
    )
    dist = (z2_ref[...] + e2_ref[...]) + dot_m2
    m = jnp.min(dist, axis=0, keepdims=True)
    idx_f = jnp.min(jnp.where(dist == m, iota_ref[...], float(dist.shape[0])),
                    axis=0, keepdims=True)
    idx = idx_f.astype(jnp.int32)
    idx_ref[...] = idx.reshape(idx.shape[1] // 128, 128)


def _compute_indices(z, embedding):
    n, d = z.shape
    ncodes = embedding.shape[0]
    z2 = jnp.sum(z ** 2, axis=1)[None, :]
    e2 = jnp.sum(embedding ** 2, axis=1)[:, None]
    iota_col = jnp.arange(ncodes, dtype=jnp.float32)[:, None]
    rows_per_blk = _COLS // 128
    idx2 = pl.pallas_call(
        _argmin_block,
        grid=(n // _COLS,),
        in_specs=[
            pl.BlockSpec((ncodes, 1), lambda i: (0, 0)),
            pl.BlockSpec((1, _COLS), lambda i: (0, i)),
            pl.BlockSpec((ncodes, 1), lambda i: (0, 0)),
            pl.BlockSpec((ncodes, d), lambda i: (0, 0)),
            pl.BlockSpec((_COLS, d), lambda i: (i, 0)),
        ],
        out_specs=pl.BlockSpec((rows_per_blk, 128), lambda i: (i, 0)),
        out_shape=jax.ShapeDtypeStruct((n // 128, 128), jnp.int32),
    )(e2, z2, iota_col, embedding, z)
    return idx2.reshape(n)


@functools.lru_cache(maxsize=None)
def _make_gather(n, d, b_per_w):
    mesh = plsc.VectorSubcoreMesh(core_axis_name="c", subcore_axis_name="s")

    @functools.partial(
        pl.kernel, mesh=mesh,
        out_type=jax.ShapeDtypeStruct((n, d), jnp.float32),
        compiler_params=pltpu.CompilerParams(use_tc_tiling_on_sc=False),
        scratch_types=[
            pltpu.VMEM((b_per_w,), jnp.int32),
            pltpu.VMEM((b_per_w, d), jnp.float32),
            pltpu.SemaphoreType.DMA,
        ],
    )
    def gather(table_hbm, idx_hbm, out_hbm, idx_v, rows_v, sem):
        wid = lax.axis_index("s") * 2 + lax.axis_index("c")
        base = wid * b_per_w
        pltpu.sync_copy(idx_hbm.at[pl.ds(base, b_per_w)], idx_v)
        pltpu.async_copy(table_hbm.at[idx_v], rows_v, sem).wait()
        pltpu.sync_copy(rows_v, out_hbm.at[pl.ds(base, b_per_w)])

    return gather


def kernel(z, embedding):
    n, d = z.shape
    idx = _compute_indices(z, embedding)
    table = embedding.astype(jnp.bfloat16).astype(jnp.float32)
    return _make_gather(n, d, n // 32)(table, idx)


# speedup vs baseline: 4.3875x; 1.0639x over previous
import functools

import jax
import jax.numpy as jnp
from jax import lax
from jax.experimental import pallas as pl
from jax.experimental.pallas import tpu as pltpu
from jax.experimental.pallas import tpu_sc as plsc

_COLS = 1024


def _argmin_block(e2_ref, z2_ref, iota_ref, emb_ref, zb_ref, idx_ref):
    emb = emb_ref[...]
    zb = zb_ref[...] * jnp.bfloat16(-2.0)
    dot_m2 = lax.dot_general(
        emb.astype(jnp.bfloat16), zb,
        (((1,), (1,)), ((), ())), preferred_element_type=jnp.float32,
    )
    dist = (z2_ref[...] + e2_ref[...]) + dot_m2
    m = jnp.min(dist, axis=0, keepdims=True)
    idx_f = jnp.min(jnp.where(dist == m, iota_ref[...], float(dist.shape[0])),
                    axis=0, keepdims=True)
    idx = idx_f.astype(jnp.int32)
    idx_ref[...] = idx.reshape(idx.shape[1] // 128, 128)


def _compute_indices(z, embedding):
    n, d = z.shape
    ncodes = embedding.shape[0]
    z2 = jnp.sum(z ** 2, axis=1)[None, :]
    e2 = jnp.sum(embedding ** 2, axis=1)[:, None]
    iota_col = jnp.arange(ncodes, dtype=jnp.float32)[:, None]
    zb = z.astype(jnp.bfloat16)
    rows_per_blk = _COLS // 128
    idx2 = pl.pallas_call(
        _argmin_block,
        grid=(n // _COLS,),
        in_specs=[
            pl.BlockSpec((ncodes, 1), lambda i: (0, 0)),
            pl.BlockSpec((1, _COLS), lambda i: (0, i)),
            pl.BlockSpec((ncodes, 1), lambda i: (0, 0)),
            pl.BlockSpec((ncodes, d), lambda i: (0, 0)),
            pl.BlockSpec((_COLS, d), lambda i: (i, 0)),
        ],
        out_specs=pl.BlockSpec((rows_per_blk, 128), lambda i: (i, 0)),
        out_shape=jax.ShapeDtypeStruct((n // 128, 128), jnp.int32),
    )(e2, z2, iota_col, embedding, zb)
    return idx2.reshape(n)


@functools.lru_cache(maxsize=None)
def _make_gather(n, d, b_per_w):
    mesh = plsc.VectorSubcoreMesh(core_axis_name="c", subcore_axis_name="s")

    @functools.partial(
        pl.kernel, mesh=mesh,
        out_type=jax.ShapeDtypeStruct((n, d), jnp.float32),
        compiler_params=pltpu.CompilerParams(use_tc_tiling_on_sc=False),
        scratch_types=[
            pltpu.VMEM((b_per_w,), jnp.int32),
            pltpu.VMEM((b_per_w, d), jnp.float32),
            pltpu.SemaphoreType.DMA,
        ],
    )
    def gather(table_hbm, idx_hbm, out_hbm, idx_v, rows_v, sem):
        wid = lax.axis_index("s") * 2 + lax.axis_index("c")
        base = wid * b_per_w
        pltpu.sync_copy(idx_hbm.at[pl.ds(base, b_per_w)], idx_v)
        pltpu.async_copy(table_hbm.at[idx_v], rows_v, sem).wait()
        pltpu.sync_copy(rows_v, out_hbm.at[pl.ds(base, b_per_w)])

    return gather


def kernel(z, embedding):
    n, d = z.shape
    idx = _compute_indices(z, embedding)
    table = embedding.astype(jnp.bfloat16).astype(jnp.float32)
    return _make_gather(n, d, n // 32)(table, idx)


# speedup vs baseline: 4.9877x; 1.1368x over previous
import functools

import jax
import jax.numpy as jnp
from jax import lax
from jax.experimental import pallas as pl
from jax.experimental.pallas import tpu as pltpu
from jax.experimental.pallas import tpu_sc as plsc

_COLS = 1024


def _argmin_block(e2_ref, z2_ref, iota_ref, emb_ref, zb_ref, idx_ref):
    emb = emb_ref[...]
    zb = zb_ref[...] * jnp.bfloat16(-2.0)
    dot_m2 = lax.dot_general(
        emb.astype(jnp.bfloat16), zb,
        (((1,), (1,)), ((), ())), preferred_element_type=jnp.float32,
    )
    ncodes, cols = dot_m2.shape
    e2 = e2_ref[...]
    iota = iota_ref[...]
    z2b = jnp.broadcast_to(z2_ref[...], (8, cols))
    val = jnp.full((8, cols), jnp.inf, jnp.float32)
    idxv = jnp.zeros((8, cols), jnp.float32)
    for k in range(ncodes // 8):
        sl = slice(8 * k, 8 * k + 8)
        distk = (z2b + e2[sl]) + dot_m2[sl, :]
        idxv = jnp.where(distk < val, jnp.broadcast_to(iota[sl], (8, cols)),
                         idxv)
        val = jnp.minimum(val, distk)
    m = jnp.min(val, axis=0, keepdims=True)
    idx_f = jnp.min(jnp.where(val == m, idxv, float(ncodes)),
                    axis=0, keepdims=True)
    idx = idx_f.astype(jnp.int32)
    idx_ref[...] = idx.reshape(idx.shape[1] // 128, 128)


def _compute_indices(z, embedding):
    n, d = z.shape
    ncodes = embedding.shape[0]
    z2 = jnp.sum(z ** 2, axis=1)[None, :]
    e2 = jnp.sum(embedding ** 2, axis=1)[:, None]
    iota_col = jnp.arange(ncodes, dtype=jnp.float32)[:, None]
    zb = z.astype(jnp.bfloat16)
    rows_per_blk = _COLS // 128
    idx2 = pl.pallas_call(
        _argmin_block,
        grid=(n // _COLS,),
        in_specs=[
            pl.BlockSpec((ncodes, 1), lambda i: (0, 0)),
            pl.BlockSpec((1, _COLS), lambda i: (0, i)),
            pl.BlockSpec((ncodes, 1), lambda i: (0, 0)),
            pl.BlockSpec((ncodes, d), lambda i: (0, 0)),
            pl.BlockSpec((_COLS, d), lambda i: (i, 0)),
        ],
        out_specs=pl.BlockSpec((rows_per_blk, 128), lambda i: (i, 0)),
        out_shape=jax.ShapeDtypeStruct((n // 128, 128), jnp.int32),
    )(e2, z2, iota_col, embedding, zb)
    return idx2.reshape(n)


@functools.lru_cache(maxsize=None)
def _make_gather(n, d, b_per_w):
    mesh = plsc.VectorSubcoreMesh(core_axis_name="c", subcore_axis_name="s")

    @functools.partial(
        pl.kernel, mesh=mesh,
        out_type=jax.ShapeDtypeStruct((n, d), jnp.float32),
        compiler_params=pltpu.CompilerParams(use_tc_tiling_on_sc=False),
        scratch_types=[
            pltpu.VMEM((b_per_w,), jnp.int32),
            pltpu.VMEM((b_per_w, d), jnp.float32),
            pltpu.SemaphoreType.DMA,
        ],
    )
    def gather(table_hbm, idx_hbm, out_hbm, idx_v, rows_v, sem):
        wid = lax.axis_index("s") * 2 + lax.axis_index("c")
        base = wid * b_per_w
        pltpu.sync_copy(idx_hbm.at[pl.ds(base, b_per_w)], idx_v)
        pltpu.async_copy(table_hbm.at[idx_v], rows_v, sem).wait()
        pltpu.sync_copy(rows_v, out_hbm.at[pl.ds(base, b_per_w)])

    return gather


def kernel(z, embedding):
    n, d = z.shape
    idx = _compute_indices(z, embedding)
    table = embedding.astype(jnp.bfloat16).astype(jnp.float32)
    return _make_gather(n, d, n // 32)(table, idx)


# speedup vs baseline: 5.1975x; 1.0421x over previous
import functools

import jax
import jax.numpy as jnp
from jax import lax
from jax.experimental import pallas as pl
from jax.experimental.pallas import tpu as pltpu
from jax.experimental.pallas import tpu_sc as plsc

_COLS = 2048


def _argmin_block(e2_ref, z2_ref, iota_ref, emb_ref, zb_ref, idx_ref):
    emb = emb_ref[...]
    zb = zb_ref[...] * jnp.bfloat16(-2.0)
    dot_m2 = lax.dot_general(
        emb.astype(jnp.bfloat16), zb,
        (((1,), (1,)), ((), ())), preferred_element_type=jnp.float32,
    )
    ncodes, cols = dot_m2.shape
    e2 = e2_ref[...]
    iota = iota_ref[...]
    z2b = jnp.broadcast_to(z2_ref[...], (8, cols))
    val = jnp.full((8, cols), jnp.inf, jnp.float32)
    idxv = jnp.zeros((8, cols), jnp.float32)
    for k in range(ncodes // 8):
        sl = slice(8 * k, 8 * k + 8)
        distk = (z2b + e2[sl]) + dot_m2[sl, :]
        idxv = jnp.where(distk < val, jnp.broadcast_to(iota[sl], (8, cols)),
                         idxv)
        val = jnp.minimum(val, distk)
    m = jnp.min(val, axis=0, keepdims=True)
    idx_f = jnp.min(jnp.where(val == m, idxv, float(ncodes)),
                    axis=0, keepdims=True)
    idx = idx_f.astype(jnp.int32)
    idx_ref[...] = idx.reshape(idx.shape[1] // 128, 128)


def _compute_indices(z, embedding):
    n, d = z.shape
    ncodes = embedding.shape[0]
    z2 = jnp.sum(z ** 2, axis=1)[None, :]
    e2 = jnp.sum(embedding ** 2, axis=1)[:, None]
    iota_col = jnp.arange(ncodes, dtype=jnp.float32)[:, None]
    zb = z.astype(jnp.bfloat16)
    rows_per_blk = _COLS // 128
    idx2 = pl.pallas_call(
        _argmin_block,
        grid=(n // _COLS,),
        in_specs=[
            pl.BlockSpec((ncodes, 1), lambda i: (0, 0)),
            pl.BlockSpec((1, _COLS), lambda i: (0, i)),
            pl.BlockSpec((ncodes, 1), lambda i: (0, 0)),
            pl.BlockSpec((ncodes, d), lambda i: (0, 0)),
            pl.BlockSpec((_COLS, d), lambda i: (i, 0)),
        ],
        out_specs=pl.BlockSpec((rows_per_blk, 128), lambda i: (i, 0)),
        out_shape=jax.ShapeDtypeStruct((n // 128, 128), jnp.int32),
    )(e2, z2, iota_col, embedding, zb)
    return idx2.reshape(n)


@functools.lru_cache(maxsize=None)
def _make_gather(n, d, b_per_w):
    mesh = plsc.VectorSubcoreMesh(core_axis_name="c", subcore_axis_name="s")

    @functools.partial(
        pl.kernel, mesh=mesh,
        out_type=jax.ShapeDtypeStruct((n, d), jnp.float32),
        compiler_params=pltpu.CompilerParams(use_tc_tiling_on_sc=False),
        scratch_types=[
            pltpu.VMEM((b_per_w,), jnp.int32),
            pltpu.VMEM((b_per_w, d), jnp.float32),
            pltpu.SemaphoreType.DMA,
        ],
    )
    def gather(table_hbm, idx_hbm, out_hbm, idx_v, rows_v, sem):
        wid = lax.axis_index("s") * 2 + lax.axis_index("c")
        base = wid * b_per_w
        pltpu.sync_copy(idx_hbm.at[pl.ds(base, b_per_w)], idx_v)
        pltpu.async_copy(table_hbm.at[idx_v], rows_v, sem).wait()
        pltpu.sync_copy(rows_v, out_hbm.at[pl.ds(base, b_per_w)])

    return gather


def kernel(z, embedding):
    n, d = z.shape
    idx = _compute_indices(z, embedding)
    table = embedding.astype(jnp.bfloat16).astype(jnp.float32)
    return _make_gather(n, d, n // 32)(table, idx)
